# Initial kernel scaffold; baseline (speedup 1.0000x reference)
import functools

import jax
import jax.numpy as jnp
from jax import lax
from jax.experimental import pallas as pl
from jax.experimental.pallas import tpu as pltpu
from jax.experimental.pallas import tpu_sc as plsc

H = 128
N_NODES = 10000
N_EDGES = 320000

NC = 2
NS = 16
NW = NC * NS

EDGES_PER_TILE = N_EDGES // NW
BLK = 80
NBLK = EDGES_PER_TILE // BLK
ROWS_PER_TILE = N_NODES // NS
CW = 16


def _sc_scatter(recv2d, e, z_sums, z_cnt, ones_blk):
    mesh = plsc.VectorSubcoreMesh(core_axis_name="c", subcore_axis_name="s")

    @functools.partial(
        pl.kernel,
        out_type=(
            jax.ShapeDtypeStruct((NC, N_NODES, H), jnp.float32),
            jax.ShapeDtypeStruct((NC, N_NODES, CW), jnp.float32),
        ),
        mesh=mesh,
        scratch_types=[
            pltpu.VMEM((NBLK, BLK), jnp.int32),
            pltpu.VMEM((BLK, H), jnp.float32),
            pltpu.VMEM((BLK, CW), jnp.float32),
            pltpu.VMEM_SHARED((N_NODES, H), jnp.float32),
            pltpu.VMEM_SHARED((N_NODES, CW), jnp.float32),
        ],
    )
    def k(recv_hbm, e_hbm, zs_hbm, zc_hbm, ones_hbm,
          sums_out, cnt_out,
          idx_v, ebuf, ones_v, sums_acc, cnt_acc):
        c = lax.axis_index("c")
        s = lax.axis_index("s")
        wid = c * NS + s

        r0 = s * ROWS_PER_TILE
        pltpu.sync_copy(zs_hbm.at[pl.ds(r0, ROWS_PER_TILE)],
                        sums_acc.at[pl.ds(r0, ROWS_PER_TILE)])
        pltpu.sync_copy(zc_hbm.at[pl.ds(r0, ROWS_PER_TILE)],
                        cnt_acc.at[pl.ds(r0, ROWS_PER_TILE)])
        pltpu.sync_copy(ones_hbm, ones_v)
        pltpu.sync_copy(recv_hbm.at[pl.ds(wid * NBLK, NBLK)], idx_v)
        plsc.subcore_barrier()

        def body(j, carry):
            base = wid * EDGES_PER_TILE + j * BLK
            pltpu.sync_copy(e_hbm.at[pl.ds(base, BLK)], ebuf)
            pltpu.sync_copy(ebuf, sums_acc.at[idx_v.at[j]], add=True)
            pltpu.sync_copy(ones_v, cnt_acc.at[idx_v.at[j]], add=True)
            return carry

        lax.fori_loop(0, NBLK, body, 0)
        plsc.subcore_barrier()

        pltpu.sync_copy(sums_acc.at[pl.ds(r0, ROWS_PER_TILE)],
                        sums_out.at[c, pl.ds(r0, ROWS_PER_TILE)])
        pltpu.sync_copy(cnt_acc.at[pl.ds(r0, ROWS_PER_TILE)],
                        cnt_out.at[c, pl.ds(r0, ROWS_PER_TILE)])

    return k(recv2d, e, z_sums, z_cnt, ones_blk)


RBLK = 1000


def _tc_mlp_body(sums_ref, cnt_ref, v_ref, w1a_ref, w1b_ref, b1_ref,
                 w2_ref, b2_ref, out_ref):
    sums = sums_ref[0] + sums_ref[1]
    cnt = cnt_ref[0][:, 0:1] + cnt_ref[1][:, 0:1]
    mean = sums / jnp.maximum(cnt, 1.0)
    h = jnp.dot(mean, w1a_ref[...], preferred_element_type=jnp.float32)
    h = h + jnp.dot(v_ref[...], w1b_ref[...], preferred_element_type=jnp.float32)
    h = jnp.maximum(h + b1_ref[...], 0.0)
    o = jnp.dot(h, w2_ref[...], preferred_element_type=jnp.float32)
    out_ref[...] = o + b2_ref[...]


def _tc_mlp(sums2, cnt2, v, W1a, W1b, b1, W2, b2):
    grid = (N_NODES // RBLK,)
    return pl.pallas_call(
        _tc_mlp_body,
        grid=grid,
        in_specs=[
            pl.BlockSpec((NC, RBLK, H), lambda i: (0, i, 0)),
            pl.BlockSpec((NC, RBLK, CW), lambda i: (0, i, 0)),
            pl.BlockSpec((RBLK, H), lambda i: (i, 0)),
            pl.BlockSpec((H, H), lambda i: (0, 0)),
            pl.BlockSpec((H, H), lambda i: (0, 0)),
            pl.BlockSpec((1, H), lambda i: (0, 0)),
            pl.BlockSpec((H, H), lambda i: (0, 0)),
            pl.BlockSpec((1, H), lambda i: (0, 0)),
        ],
        out_specs=pl.BlockSpec((RBLK, H), lambda i: (i, 0)),
        out_shape=jax.ShapeDtypeStruct((N_NODES, H), jnp.float32),
    )(sums2, cnt2, v, W1a, W1b, b1, W2, b2)


def kernel(v, edge_index, e, W1, b1, W2, b2):
    recv = edge_index[1].astype(jnp.int32).reshape(NW * NBLK, BLK)
    z_sums = jnp.zeros((N_NODES, H), jnp.float32)
    z_cnt = jnp.zeros((N_NODES, CW), jnp.float32)
    ones_blk = jnp.ones((BLK, CW), jnp.float32)

    sums2, cnt2 = _sc_scatter(recv, e, z_sums, z_cnt, ones_blk)

    W1a = W1[:H]
    W1b = W1[H:]
    return _tc_mlp(sums2, cnt2, v, W1a, W1b,
                   b1.reshape(1, H), W2, b2.reshape(1, H))


# speedup vs baseline: 4.4292x; 4.4292x over previous
import functools

import jax
import jax.numpy as jnp
from jax import lax
from jax.experimental import pallas as pl
from jax.experimental.pallas import tpu as pltpu
from jax.experimental.pallas import tpu_sc as plsc

H = 128
N_NODES = 10000
N_EDGES = 320000

NC = 2
NS = 16
NW = NC * NS

EDGES_PER_TILE = N_EDGES // NW
BLK = 80
NBLK = EDGES_PER_TILE // BLK
N_ACC = 10240
ROWS_PER_TILE = N_ACC // NS

MESH = plsc.VectorSubcoreMesh(core_axis_name="c", subcore_axis_name="s")


def _sc_sums(recv3d, e, zeros_acc):
    @functools.partial(
        pl.kernel,
        out_type=jax.ShapeDtypeStruct((NC * N_ACC, H), jnp.float32),
        mesh=MESH,
        scratch_types=[
            pltpu.VMEM((NBLK, BLK), jnp.int32),
            pltpu.VMEM((BLK, H), jnp.float32),
            pltpu.VMEM_SHARED((N_ACC, H), jnp.float32),
        ],
    )
    def k(recv_hbm, e_hbm, z_hbm, out_hbm, idx_v, ebuf, acc):
        c = lax.axis_index("c")
        s = lax.axis_index("s")
        w = c * NS + s
        r0 = s * ROWS_PER_TILE
        pltpu.sync_copy(z_hbm.at[pl.ds(r0, ROWS_PER_TILE)],
                        acc.at[pl.ds(r0, ROWS_PER_TILE)])
        pltpu.sync_copy(recv_hbm.at[w], idx_v)
        plsc.subcore_barrier()

        def body(j, carry):
            base = w * EDGES_PER_TILE + j * BLK
            pltpu.sync_copy(e_hbm.at[pl.ds(base, BLK)], ebuf)
            pltpu.sync_copy(ebuf, acc.at[idx_v.at[j]], add=True)
            return carry

        lax.fori_loop(0, NBLK, body, 0)
        plsc.subcore_barrier()
        pltpu.sync_copy(acc.at[pl.ds(r0, ROWS_PER_TILE)],
                        out_hbm.at[pl.ds(c * N_ACC + r0, ROWS_PER_TILE)])

    return k(recv3d, e, zeros_acc)


def _sc_counts(recv3d, zeros_acc, ones_blk):
    @functools.partial(
        pl.kernel,
        out_type=jax.ShapeDtypeStruct((NC * N_ACC, H), jnp.float32),
        mesh=MESH,
        scratch_types=[
            pltpu.VMEM((NBLK, BLK), jnp.int32),
            pltpu.VMEM((BLK, H), jnp.float32),
            pltpu.VMEM_SHARED((N_ACC, H), jnp.float32),
        ],
    )
    def k(recv_hbm, z_hbm, ones_hbm, out_hbm, idx_v, ones_v, acc):
        c = lax.axis_index("c")
        s = lax.axis_index("s")
        w = c * NS + s
        r0 = s * ROWS_PER_TILE
        pltpu.sync_copy(z_hbm.at[pl.ds(r0, ROWS_PER_TILE)],
                        acc.at[pl.ds(r0, ROWS_PER_TILE)])
        pltpu.sync_copy(ones_hbm, ones_v)
        pltpu.sync_copy(recv_hbm.at[w], idx_v)
        plsc.subcore_barrier()

        def body(j, carry):
            pltpu.sync_copy(ones_v, acc.at[idx_v.at[j]], add=True)
            return carry

        lax.fori_loop(0, NBLK, body, 0)
        plsc.subcore_barrier()
        pltpu.sync_copy(acc.at[pl.ds(r0, ROWS_PER_TILE)],
                        out_hbm.at[pl.ds(c * N_ACC + r0, ROWS_PER_TILE)])

    return k(recv3d, zeros_acc, ones_blk)


RBLK = 1024


def _tc_mlp_body(s0_ref, s1_ref, c0_ref, c1_ref, v_ref, w1a_ref, w1b_ref,
                 b1_ref, w2_ref, b2_ref, out_ref):
    sums = s0_ref[0] + s1_ref[0]
    cnt = c0_ref[0][:, 0:1] + c1_ref[0][:, 0:1]
    mean = sums / jnp.maximum(cnt, 1.0)
    h = jnp.dot(mean, w1a_ref[...], preferred_element_type=jnp.float32)
    h = h + jnp.dot(v_ref[...], w1b_ref[...], preferred_element_type=jnp.float32)
    h = jnp.maximum(h + b1_ref[...], 0.0)
    o = jnp.dot(h, w2_ref[...], preferred_element_type=jnp.float32)
    out_ref[...] = o + b2_ref[...]


def _tc_mlp(sums3, cnt3, v, W1a, W1b, b1, W2, b2):
    grid = (N_ACC // RBLK,)
    part = pl.BlockSpec((1, RBLK, H), lambda i: (0, i, 0))
    part1 = pl.BlockSpec((1, RBLK, H), lambda i: (1, i, 0))
    full = pl.BlockSpec((H, H), lambda i: (0, 0))
    bias = pl.BlockSpec((1, H), lambda i: (0, 0))
    return pl.pallas_call(
        _tc_mlp_body,
        grid=grid,
        in_specs=[part, part1, part, part1,
                  pl.BlockSpec((RBLK, H), lambda i: (i, 0)),
                  full, full, bias, full, bias],
        out_specs=pl.BlockSpec((RBLK, H), lambda i: (i, 0)),
        out_shape=jax.ShapeDtypeStruct((N_NODES, H), jnp.float32),
    )(sums3, sums3, cnt3, cnt3, v, W1a, W1b, b1, W2, b2)


def kernel(v, edge_index, e, W1, b1, W2, b2):
    recv = edge_index[1].astype(jnp.int32).reshape(NW, NBLK, BLK)
    z_acc = jnp.zeros((N_ACC, H), jnp.float32)
    ones_blk = jnp.ones((BLK, H), jnp.float32)

    sums = _sc_sums(recv, e, z_acc).reshape(NC, N_ACC, H)
    cnt = _sc_counts(recv, z_acc, ones_blk).reshape(NC, N_ACC, H)

    return _tc_mlp(sums, cnt, v, W1[:H], W1[H:],
                   b1.reshape(1, H), W2, b2.reshape(1, H))


# speedup vs baseline: 8.0153x; 1.8096x over previous
import functools

import jax
import jax.numpy as jnp
from jax import lax
from jax.experimental import pallas as pl
from jax.experimental.pallas import tpu as pltpu
from jax.experimental.pallas import tpu_sc as plsc

H = 128
N_NODES = 10000
N_EDGES = 320000

NC = 2
NS = 16
NW = NC * NS

EDGES_PER_TILE = N_EDGES // NW
BLK = 80
NBLK = EDGES_PER_TILE // BLK
N_ACC = 10240
ROWS_PER_TILE = N_ACC // NS
CROWS = N_ACC // H

MESH = plsc.VectorSubcoreMesh(core_axis_name="c", subcore_axis_name="s")


def _sc_fused(recv3d, e, z_acc, iota_c):
    @functools.partial(
        pl.kernel,
        out_type=(
            jax.ShapeDtypeStruct((NC * N_ACC, H), jnp.float32),
            jax.ShapeDtypeStruct((NC * CROWS, H), jnp.float32),
        ),
        mesh=MESH,
        compiler_params=pltpu.CompilerParams(needs_layout_passes=False),
        scratch_types=[
            pltpu.VMEM((NBLK, BLK), jnp.int32),
            pltpu.VMEM((BLK, H), jnp.float32),
            pltpu.VMEM((BLK, H), jnp.float32),
            pltpu.VMEM((CROWS, H), jnp.float32),
            pltpu.VMEM((CROWS,), jnp.int32),
            pltpu.VMEM_SHARED((N_ACC, H), jnp.float32),
            pltpu.VMEM_SHARED((CROWS, H), jnp.float32),
            pltpu.SemaphoreType.DMA,
            pltpu.SemaphoreType.DMA,
        ],
    )
    def k(recv_hbm, e_hbm, z_hbm, iota_hbm,
          sums_out, cnt_out,
          idx_v, ebuf0, ebuf1, cnt2d, iota_v, acc, acc_c, sem0, sem1):
        c = lax.axis_index("c")
        s = lax.axis_index("s")
        w = c * NS + s
        r0 = s * ROWS_PER_TILE
        ebase = w * EDGES_PER_TILE

        pltpu.sync_copy(z_hbm.at[pl.ds(r0, ROWS_PER_TILE)],
                        acc.at[pl.ds(r0, ROWS_PER_TILE)])
        pltpu.sync_copy(z_hbm.at[pl.ds(0, CROWS)], cnt2d)
        pltpu.sync_copy(iota_hbm, iota_v)
        pltpu.sync_copy(recv_hbm.at[w], idx_v)

        @pl.when(s == 0)
        def _():
            pltpu.sync_copy(z_hbm.at[pl.ds(0, CROWS)], acc_c)

        plsc.subcore_barrier()

        ones16 = jnp.ones((16,), jnp.float32)

        def count_block(j):
            for kk in range(BLK // 16):
                idx16 = idx_v[j, pl.ds(kk * 16, 16)]
                row = lax.shift_right_logical(idx16, 7)
                col = lax.bitwise_and(idx16, 127)
                plsc.addupdate_scatter(cnt2d, [row, col], ones16)

        pltpu.async_copy(e_hbm.at[pl.ds(ebase, BLK)], ebuf0, sem0)

        def body(jj, carry):
            j0 = 2 * jj
            j1 = 2 * jj + 1
            pltpu.async_copy(e_hbm.at[pl.ds(ebase + j1 * BLK, BLK)], ebuf1, sem1)
            count_block(j0)
            pltpu.make_async_copy(e_hbm.at[pl.ds(ebase, BLK)], ebuf0, sem0).wait()
            pltpu.sync_copy(ebuf0, acc.at[idx_v.at[j0]], add=True)
            pltpu.async_copy(e_hbm.at[pl.ds(ebase + (j0 + 2) * BLK, BLK)], ebuf0, sem0)
            count_block(j1)
            pltpu.make_async_copy(e_hbm.at[pl.ds(ebase, BLK)], ebuf1, sem1).wait()
            pltpu.sync_copy(ebuf1, acc.at[idx_v.at[j1]], add=True)
            return carry

        lax.fori_loop(0, (NBLK - 1) // 2, body, 0)
        count_block(NBLK - 1)
        pltpu.make_async_copy(e_hbm.at[pl.ds(ebase, BLK)], ebuf0, sem0).wait()
        pltpu.sync_copy(ebuf0, acc.at[idx_v.at[NBLK - 1]], add=True)

        plsc.subcore_barrier()
        pltpu.sync_copy(cnt2d, acc_c.at[iota_v], add=True)
        plsc.subcore_barrier()

        pltpu.sync_copy(acc.at[pl.ds(r0, ROWS_PER_TILE)],
                        sums_out.at[pl.ds(c * N_ACC + r0, ROWS_PER_TILE)])

        @pl.when(s == 0)
        def _():
            pltpu.sync_copy(acc_c, cnt_out.at[pl.ds(c * CROWS, CROWS)])

    return k(recv3d, e, z_acc, iota_c)


RBLK = 1024


def _tc_mlp_body(s0_ref, s1_ref, c0_ref, c1_ref, v_ref, w1a_ref, w1b_ref,
                 b1_ref, w2_ref, b2_ref, out_ref):
    sums = s0_ref[0] + s1_ref[0]
    cnt = c0_ref[0] + c1_ref[0]
    mean = sums / jnp.maximum(cnt, 1.0)
    h = jnp.dot(mean, w1a_ref[...], preferred_element_type=jnp.float32)
    h = h + jnp.dot(v_ref[...], w1b_ref[...], preferred_element_type=jnp.float32)
    h = jnp.maximum(h + b1_ref[...], 0.0)
    o = jnp.dot(h, w2_ref[...], preferred_element_type=jnp.float32)
    out_ref[...] = o + b2_ref[...]


def _tc_mlp(sums3, cnt3, v, W1a, W1b, b1, W2, b2):
    grid = (N_ACC // RBLK,)
    part0 = pl.BlockSpec((1, RBLK, H), lambda i: (0, i, 0))
    part1 = pl.BlockSpec((1, RBLK, H), lambda i: (1, i, 0))
    col0 = pl.BlockSpec((1, RBLK, 1), lambda i: (0, i, 0))
    col1 = pl.BlockSpec((1, RBLK, 1), lambda i: (1, i, 0))
    full = pl.BlockSpec((H, H), lambda i: (0, 0))
    bias = pl.BlockSpec((1, H), lambda i: (0, 0))
    return pl.pallas_call(
        _tc_mlp_body,
        grid=grid,
        in_specs=[part0, part1, col0, col1,
                  pl.BlockSpec((RBLK, H), lambda i: (i, 0)),
                  full, full, bias, full, bias],
        out_specs=pl.BlockSpec((RBLK, H), lambda i: (i, 0)),
        out_shape=jax.ShapeDtypeStruct((N_NODES, H), jnp.float32),
    )(sums3, sums3, cnt3, cnt3, v, W1a, W1b, b1, W2, b2)


def kernel(v, edge_index, e, W1, b1, W2, b2):
    recv = edge_index[1].astype(jnp.int32).reshape(NW, NBLK, BLK)
    z_acc = jnp.zeros((N_ACC, H), jnp.float32)
    iota_c = jnp.arange(CROWS, dtype=jnp.int32)

    sums, cnt = _sc_fused(recv, e, z_acc, iota_c)
    sums = sums.reshape(NC, N_ACC, H)
    cnt_col = cnt.reshape(NC, N_ACC)[:, :, None]

    return _tc_mlp(sums, cnt_col, v, W1[:H], W1[H:],
                   b1.reshape(1, H), W2, b2.reshape(1, H))


# speedup vs baseline: 9.1531x; 1.1419x over previous
import functools

import jax
import jax.numpy as jnp
import numpy as np
from jax import lax
from jax.experimental import pallas as pl
from jax.experimental.pallas import tpu as pltpu
from jax.experimental.pallas import tpu_sc as plsc

H = 128
N_NODES = 10000
N_EDGES = 320000

NC = 2
NS = 16
NW = NC * NS

EDGES_PER_TILE = N_EDGES // NW
BLK = 80
NBLK = EDGES_PER_TILE // BLK
N_ACC = 10240
ROWS_PER_TILE = N_ACC // NS
CROWS = N_ACC // H
ZCH = ROWS_PER_TILE // CROWS

MESH = plsc.VectorSubcoreMesh(core_axis_name="c", subcore_axis_name="s")


def _sc_fused(recv3d, e):
    @functools.partial(
        pl.kernel,
        out_type=(
            jax.ShapeDtypeStruct((NC * N_ACC, H), jnp.float32),
            jax.ShapeDtypeStruct((NC * CROWS, H), jnp.float32),
        ),
        mesh=MESH,
        compiler_params=pltpu.CompilerParams(needs_layout_passes=False),
        scratch_types=[
            pltpu.VMEM((NBLK, BLK), jnp.int32),
            pltpu.VMEM((BLK, H), jnp.float32),
            pltpu.VMEM((BLK, H), jnp.float32),
            pltpu.VMEM((CROWS, H), jnp.float32),
            pltpu.VMEM((CROWS,), jnp.int32),
            pltpu.VMEM_SHARED((N_ACC, H), jnp.float32),
            pltpu.VMEM_SHARED((CROWS, H), jnp.float32),
            pltpu.SemaphoreType.DMA,
            pltpu.SemaphoreType.DMA,
        ],
    )
    def k(recv_hbm, e_hbm,
          sums_out, cnt_out,
          idx_v, ebuf0, ebuf1, cnt2d, iota_v, acc, acc_c, sem0, sem1):
        c = lax.axis_index("c")
        s = lax.axis_index("s")
        w = c * NS + s
        r0 = s * ROWS_PER_TILE
        ebase = w * EDGES_PER_TILE

        pltpu.async_copy(recv_hbm.at[w], idx_v, sem1)

        z16 = jnp.zeros((16,), jnp.float32)
        for kk in range(CROWS // 16):
            iota_v[pl.ds(kk * 16, 16)] = lax.iota(jnp.int32, 16) + 16 * kk

        def zrow(j, carry):
            for kk in range(H // 16):
                cnt2d[j, pl.ds(kk * 16, 16)] = z16
            return carry

        lax.fori_loop(0, CROWS, zrow, 0)

        for zz in range(ZCH):
            pltpu.sync_copy(cnt2d, acc.at[pl.ds(r0 + zz * CROWS, CROWS)])

        @pl.when(s == 0)
        def _():
            pltpu.sync_copy(cnt2d, acc_c)

        pltpu.make_async_copy(recv_hbm.at[w], idx_v, sem1).wait()
        plsc.subcore_barrier()

        ones16 = jnp.ones((16,), jnp.float32)

        def count_block(j):
            for kk in range(BLK // 16):
                idx16 = idx_v[j, pl.ds(kk * 16, 16)]
                row = lax.shift_right_logical(idx16, 7)
                col = lax.bitwise_and(idx16, 127)
                plsc.addupdate_scatter(cnt2d, [row, col], ones16)

        pltpu.async_copy(e_hbm.at[pl.ds(ebase, BLK)], ebuf0, sem0)

        def body(jj, carry):
            j0 = 2 * jj
            j1 = 2 * jj + 1
            pltpu.async_copy(e_hbm.at[pl.ds(ebase + j1 * BLK, BLK)], ebuf1, sem1)
            count_block(j0)
            pltpu.make_async_copy(e_hbm.at[pl.ds(ebase, BLK)], ebuf0, sem0).wait()
            pltpu.sync_copy(ebuf0, acc.at[idx_v.at[j0]], add=True)
            pltpu.async_copy(e_hbm.at[pl.ds(ebase + (j0 + 2) * BLK, BLK)], ebuf0, sem0)
            count_block(j1)
            pltpu.make_async_copy(e_hbm.at[pl.ds(ebase, BLK)], ebuf1, sem1).wait()
            pltpu.sync_copy(ebuf1, acc.at[idx_v.at[j1]], add=True)
            return carry

        lax.fori_loop(0, (NBLK - 1) // 2, body, 0)
        count_block(NBLK - 1)
        pltpu.make_async_copy(e_hbm.at[pl.ds(ebase, BLK)], ebuf0, sem0).wait()
        pltpu.sync_copy(ebuf0, acc.at[idx_v.at[NBLK - 1]], add=True)

        plsc.subcore_barrier()
        pltpu.sync_copy(cnt2d, acc_c.at[iota_v], add=True)
        plsc.subcore_barrier()

        pltpu.sync_copy(acc.at[pl.ds(r0, ROWS_PER_TILE)],
                        sums_out.at[pl.ds(c * N_ACC + r0, ROWS_PER_TILE)])

        @pl.when(s == 0)
        def _():
            pltpu.sync_copy(acc_c, cnt_out.at[pl.ds(c * CROWS, CROWS)])

    return k(recv3d, e)


RBLK = 1024
CR_B = RBLK // H


def _tc_mlp_body(s0_ref, s1_ref, c0_ref, c1_ref, p_ref, b_ref, v_ref,
                 w1a_ref, w1b_ref, b1_ref, w2_ref, b2_ref, out_ref):
    sums = s0_ref[0] + s1_ref[0]
    cimg = c0_ref[0] + c1_ref[0]
    expanded = jnp.dot(p_ref[...], cimg,
                       preferred_element_type=jnp.float32)
    cnt = jnp.sum(expanded * b_ref[...], axis=1, keepdims=True)
    mean = sums / jnp.maximum(cnt, 1.0)
    h = jnp.dot(mean, w1a_ref[...], preferred_element_type=jnp.float32)
    h = h + jnp.dot(v_ref[...], w1b_ref[...], preferred_element_type=jnp.float32)
    h = jnp.maximum(h + b1_ref[...], 0.0)
    o = jnp.dot(h, w2_ref[...], preferred_element_type=jnp.float32)
    out_ref[...] = o + b2_ref[...]


def _tc_mlp(sums3, cnt3, v, W1a, W1b, b1, W2, b2):
    r = np.arange(RBLK)
    P = (r[:, None] >> 7 == np.arange(CR_B)[None, :]).astype(np.float32)
    B = ((r[:, None] & 127) == np.arange(H)[None, :]).astype(np.float32)
    grid = (N_ACC // RBLK,)
    part0 = pl.BlockSpec((1, RBLK, H), lambda i: (0, i, 0))
    part1 = pl.BlockSpec((1, RBLK, H), lambda i: (1, i, 0))
    cim0 = pl.BlockSpec((1, CR_B, H), lambda i: (0, i, 0))
    cim1 = pl.BlockSpec((1, CR_B, H), lambda i: (1, i, 0))
    full = pl.BlockSpec((H, H), lambda i: (0, 0))
    bias = pl.BlockSpec((1, H), lambda i: (0, 0))
    return pl.pallas_call(
        _tc_mlp_body,
        grid=grid,
        in_specs=[part0, part1, cim0, cim1,
                  pl.BlockSpec((RBLK, CR_B), lambda i: (0, 0)),
                  pl.BlockSpec((RBLK, H), lambda i: (0, 0)),
                  pl.BlockSpec((RBLK, H), lambda i: (i, 0)),
                  full, full, bias, full, bias],
        out_specs=pl.BlockSpec((RBLK, H), lambda i: (i, 0)),
        out_shape=jax.ShapeDtypeStruct((N_NODES, H), jnp.float32),
    )(sums3, sums3, cnt3, cnt3, jnp.asarray(P), jnp.asarray(B),
      v, W1a, W1b, b1, W2, b2)


def kernel(v, edge_index, e, W1, b1, W2, b2):
    recv = edge_index[1].astype(jnp.int32).reshape(NW, NBLK, BLK)

    sums, cnt = _sc_fused(recv, e)
    sums = sums.reshape(NC, N_ACC, H)
    cimg = cnt.reshape(NC, CROWS, H)

    return _tc_mlp(sums, cimg, v, W1[:H], W1[H:],
                   b1.reshape(1, H), W2, b2.reshape(1, H))
